# baseline (speedup 1.0000x reference)
import functools

import jax
import jax.numpy as jnp
from jax import lax
from jax.experimental import pallas as pl
from jax.experimental.pallas import tpu as pltpu
from jax.experimental.pallas import tpu_sc as plsc

VOCAB = 100000
D = 64
B = 1024
CTX = 20

NC = 2
NS = 16
NW = NC * NS
BPW = B // NW
LANES = 16
KV = D // LANES

VBLK = 2048
NVBLK = (VOCAB + VBLK - 1) // VBLK
TAIL = VOCAB - (NVBLK - 1) * VBLK
VPAD = NVBLK * VBLK
NBUF = 4


@functools.cache
def _make_gather_mean():
    mesh = plsc.VectorSubcoreMesh(core_axis_name="c", subcore_axis_name="s")

    @functools.partial(
        pl.kernel,
        mesh=mesh,
        out_type=jax.ShapeDtypeStruct((B, D), jnp.float32),
        scratch_types=[
            pltpu.VMEM((CTX, BPW), jnp.int32),
            pltpu.VMEM((CTX, BPW, D), jnp.float32),
            pltpu.VMEM((BPW, D), jnp.float32),
            pltpu.SemaphoreType.DMA,
        ],
        compiler_params=pltpu.CompilerParams(use_tc_tiling_on_sc=False),
    )
    def _gather_mean(idx_hbm, table_hbm, out_hbm, idx_v, rows_v, acc_v, sem):
        wid = lax.axis_index("s") * NC + lax.axis_index("c")
        pltpu.sync_copy(idx_hbm.at[wid], idx_v)
        copies = []
        for j in range(CTX):
            copies.append(
                pltpu.async_copy(table_hbm.at[idx_v.at[j]], rows_v.at[j], sem))
        for c in copies:
            c.wait()

        def row_body(r, carry):
            for k in range(KV):
                sl = pl.ds(k * LANES, LANES)
                acc = rows_v[0, r, sl]
                for j in range(1, CTX):
                    acc = acc + rows_v[j, r, sl]
                acc_v[r, sl] = acc * (1.0 / CTX)
            return carry

        lax.fori_loop(0, BPW, row_body, 0)
        pltpu.sync_copy(acc_v, out_hbm.at[pl.ds(wid * BPW, BPW)])

    return _gather_mean


def _mm_body(mean_ref, w_ref, b_ref, out_hbm, acc_ref, tail_ref, sems, tail_sem):
    i = pl.program_id(0)
    slot = lax.rem(i, NBUF)

    @pl.when(i >= NBUF)
    def _():
        prev = i - NBUF
        pltpu.make_async_copy(
            acc_ref.at[slot],
            out_hbm.at[:, pl.ds(prev * VBLK, VBLK)],
            sems.at[slot],
        ).wait()

    @pl.when(i < NVBLK - 1)
    def _():
        acc_ref[slot] = lax.dot_general(
            mean_ref[...], w_ref[...],
            dimension_numbers=(((1,), (1,)), ((), ())),
            preferred_element_type=jnp.float32,
        ) + b_ref[0]
        pltpu.make_async_copy(
            acc_ref.at[slot],
            out_hbm.at[:, pl.ds(i * VBLK, VBLK)],
            sems.at[slot],
        ).start()

    @pl.when(i == NVBLK - 1)
    def _():
        tail_ref[...] = lax.dot_general(
            mean_ref[...], w_ref[pl.ds(0, TAIL), :],
            dimension_numbers=(((1,), (1,)), ((), ())),
            preferred_element_type=jnp.float32,
        ) + b_ref[0][:, :TAIL]
        pltpu.make_async_copy(
            tail_ref,
            out_hbm.at[:, pl.ds((NVBLK - 1) * VBLK, TAIL)],
            tail_sem,
        ).start()
        for k in range(NBUF - 1):
            step = NVBLK - NBUF + k
            pltpu.make_async_copy(
                acc_ref.at[step % NBUF],
                out_hbm.at[:, pl.ds(step * VBLK, VBLK)],
                sems.at[step % NBUF],
            ).wait()
        pltpu.make_async_copy(
            tail_ref,
            out_hbm.at[:, pl.ds((NVBLK - 1) * VBLK, TAIL)],
            tail_sem,
        ).wait()


@functools.cache
def _make_matmul():
    return pl.pallas_call(
        _mm_body,
        grid=(NVBLK,),
        in_specs=[
            pl.BlockSpec((B, D), lambda i: (0, 0)),
            pl.BlockSpec((VBLK, D), lambda i: (i, 0)),
            pl.BlockSpec((1, 1, VBLK), lambda i: (i, 0, 0)),
        ],
        out_specs=pl.BlockSpec(memory_space=pltpu.HBM),
        out_shape=jax.ShapeDtypeStruct((B, VOCAB), jnp.float32),
        scratch_shapes=[
            pltpu.VMEM((NBUF, B, VBLK), jnp.float32),
            pltpu.VMEM((B, TAIL), jnp.float32),
            pltpu.SemaphoreType.DMA((NBUF,)),
            pltpu.SemaphoreType.DMA,
        ],
        compiler_params=pltpu.CompilerParams(
            dimension_semantics=("arbitrary",),
        ),
    )


def kernel(context_window, emb_table, W, b):
    idx = context_window.astype(jnp.int32).reshape(NW, BPW, CTX).transpose(0, 2, 1)
    mean_emb = _make_gather_mean()(idx, emb_table)
    b_pad = jnp.pad(b, (0, VPAD - VOCAB)).reshape(NVBLK, 1, VBLK)
    return _make_matmul()(mean_emb, W, b_pad)


# speedup vs baseline: 1.0067x; 1.0067x over previous
import functools

import jax
import jax.numpy as jnp
from jax import lax
from jax.experimental import pallas as pl
from jax.experimental.pallas import tpu as pltpu
from jax.experimental.pallas import tpu_sc as plsc

VOCAB = 100000
D = 64
B = 1024
CTX = 20

NC = 2
NS = 16
NW = NC * NS
BPW = B // NW
LANES = 16
KV = D // LANES

VBLK = 2048
NVBLK = (VOCAB + VBLK - 1) // VBLK
TAIL = VOCAB - (NVBLK - 1) * VBLK
VPAD = NVBLK * VBLK
NBUF = 4


@functools.cache
def _make_gather_mean():
    mesh = plsc.VectorSubcoreMesh(core_axis_name="c", subcore_axis_name="s")

    @functools.partial(
        pl.kernel,
        mesh=mesh,
        out_type=jax.ShapeDtypeStruct((B, D), jnp.float32),
        scratch_types=[
            pltpu.VMEM((CTX, BPW), jnp.int32),
            pltpu.VMEM((CTX, BPW, D), jnp.float32),
            pltpu.VMEM((BPW, D), jnp.float32),
            pltpu.SemaphoreType.DMA,
        ],
        compiler_params=pltpu.CompilerParams(use_tc_tiling_on_sc=False),
    )
    def _gather_mean(idx_hbm, table_hbm, out_hbm, idx_v, rows_v, acc_v, sem):
        wid = lax.axis_index("s") * NC + lax.axis_index("c")
        pltpu.sync_copy(idx_hbm.at[wid], idx_v)
        copies = []
        for j in range(CTX):
            copies.append(
                pltpu.async_copy(table_hbm.at[idx_v.at[j]], rows_v.at[j], sem))
        for c in copies:
            c.wait()

        def row_body(r, carry):
            for k in range(KV):
                sl = pl.ds(k * LANES, LANES)
                acc = rows_v[0, r, sl]
                for j in range(1, CTX):
                    acc = acc + rows_v[j, r, sl]
                acc_v[r, sl] = acc * (1.0 / CTX)
            return carry

        lax.fori_loop(0, BPW, row_body, 0)
        pltpu.sync_copy(acc_v, out_hbm.at[pl.ds(wid * BPW, BPW)])

    return _gather_mean


NQ = 4
RQ = B // NQ


def _blk_copy(acc_ref, out_hbm, sems, slot, step):
    for q in range(NQ):
        yield pltpu.make_async_copy(
            acc_ref.at[slot, pl.ds(q * RQ, RQ)],
            out_hbm.at[pl.ds(q * RQ, RQ), pl.ds(step * VBLK, VBLK)],
            sems.at[slot, q],
        )


def _mm_body(mean_ref, w_ref, b_ref, out_hbm, acc_ref, tail_ref, sems, tail_sem):
    i = pl.program_id(0)
    slot = lax.rem(i, NBUF)

    @pl.when(i >= NBUF)
    def _():
        prev = i - NBUF
        for c in _blk_copy(acc_ref, out_hbm, sems, slot, prev):
            c.wait()

    @pl.when(i < NVBLK - 1)
    def _():
        acc_ref[slot] = lax.dot_general(
            mean_ref[...], w_ref[...],
            dimension_numbers=(((1,), (1,)), ((), ())),
            preferred_element_type=jnp.float32,
        ) + b_ref[0]
        for c in _blk_copy(acc_ref, out_hbm, sems, slot, i):
            c.start()

    @pl.when(i == NVBLK - 1)
    def _():
        tail_ref[...] = lax.dot_general(
            mean_ref[...], w_ref[pl.ds(0, TAIL), :],
            dimension_numbers=(((1,), (1,)), ((), ())),
            preferred_element_type=jnp.float32,
        ) + b_ref[0][:, :TAIL]
        pltpu.make_async_copy(
            tail_ref,
            out_hbm.at[:, pl.ds((NVBLK - 1) * VBLK, TAIL)],
            tail_sem,
        ).start()
        for k in range(NBUF - 1):
            step = NVBLK - NBUF + k
            for c in _blk_copy(acc_ref, out_hbm, sems, step % NBUF, step):
                c.wait()
        pltpu.make_async_copy(
            tail_ref,
            out_hbm.at[:, pl.ds((NVBLK - 1) * VBLK, TAIL)],
            tail_sem,
        ).wait()


@functools.cache
def _make_matmul():
    return pl.pallas_call(
        _mm_body,
        grid=(NVBLK,),
        in_specs=[
            pl.BlockSpec((B, D), lambda i: (0, 0)),
            pl.BlockSpec((VBLK, D), lambda i: (i, 0)),
            pl.BlockSpec((1, 1, VBLK), lambda i: (i, 0, 0)),
        ],
        out_specs=pl.BlockSpec(memory_space=pltpu.HBM),
        out_shape=jax.ShapeDtypeStruct((B, VOCAB), jnp.float32),
        scratch_shapes=[
            pltpu.VMEM((NBUF, B, VBLK), jnp.float32),
            pltpu.VMEM((B, TAIL), jnp.float32),
            pltpu.SemaphoreType.DMA((NBUF, NQ)),
            pltpu.SemaphoreType.DMA,
        ],
        compiler_params=pltpu.CompilerParams(
            dimension_semantics=("arbitrary",),
        ),
    )


def kernel(context_window, emb_table, W, b):
    idx = context_window.astype(jnp.int32).reshape(NW, BPW, CTX).transpose(0, 2, 1)
    mean_emb = _make_gather_mean()(idx, emb_table)
    b_pad = jnp.pad(b, (0, VPAD - VOCAB)).reshape(NVBLK, 1, VBLK)
    return _make_matmul()(mean_emb, W, b_pad)


# speedup vs baseline: 2.8066x; 2.7880x over previous
import functools

import jax
import jax.numpy as jnp
from jax import lax
from jax.experimental import pallas as pl
from jax.experimental.pallas import tpu as pltpu
from jax.experimental.pallas import tpu_sc as plsc

VOCAB = 100000
D = 64
B = 1024
CTX = 20

NC = 2
NS = 16
NW = NC * NS
BPW = B // NW
LANES = 16
KV = D // LANES

VBLK = 2048
NVBLK = (VOCAB + VBLK - 1) // VBLK
TAIL = VOCAB - (NVBLK - 1) * VBLK
VPAD = NVBLK * VBLK
NBUF = 4


@functools.cache
def _make_gather_mean():
    mesh = plsc.VectorSubcoreMesh(core_axis_name="c", subcore_axis_name="s")

    @functools.partial(
        pl.kernel,
        mesh=mesh,
        out_type=jax.ShapeDtypeStruct((B, D), jnp.float32),
        scratch_types=[
            pltpu.VMEM((CTX, BPW), jnp.int32),
            pltpu.VMEM((CTX, BPW, D), jnp.float32),
            pltpu.VMEM((BPW, D), jnp.float32),
            pltpu.SemaphoreType.DMA,
        ],
        compiler_params=pltpu.CompilerParams(use_tc_tiling_on_sc=False),
    )
    def _gather_mean(idx_hbm, table_hbm, out_hbm, idx_v, rows_v, acc_v, sem):
        wid = lax.axis_index("s") * NC + lax.axis_index("c")
        pltpu.sync_copy(idx_hbm.at[wid], idx_v)
        copies = []
        for j in range(CTX):
            copies.append(
                pltpu.async_copy(table_hbm.at[idx_v.at[j]], rows_v.at[j], sem))
        for c in copies:
            c.wait()

        def row_body(r, carry):
            for k in range(KV):
                sl = pl.ds(k * LANES, LANES)
                acc = rows_v[0, r, sl]
                for j in range(1, CTX):
                    acc = acc + rows_v[j, r, sl]
                acc_v[r, sl] = acc * (1.0 / CTX)
            return carry

        lax.fori_loop(0, BPW, row_body, 0)
        pltpu.sync_copy(acc_v, out_hbm.at[pl.ds(wid * BPW, BPW)])

    return _gather_mean


def _mm_body(mean_ref, wt_ref, b_ref, out_hbm, acc_ref, tail_ref, sems, tail_sem):
    i = pl.program_id(0)
    slot = lax.rem(i, NBUF)

    @pl.when(i >= NBUF)
    def _():
        prev = i - NBUF
        pltpu.make_async_copy(
            acc_ref.at[slot],
            out_hbm.at[pl.ds(prev * VBLK, VBLK), :],
            sems.at[slot],
        ).wait()

    waug = jnp.concatenate([wt_ref[...], b_ref[0]], axis=0)
    maug = jnp.concatenate(
        [mean_ref[...], jnp.ones((B, 1), jnp.float32)], axis=1)
    blk = lax.dot_general(
        waug, maug,
        dimension_numbers=(((0,), (1,)), ((), ())),
        preferred_element_type=jnp.float32,
    )

    @pl.when(i < NVBLK - 1)
    def _():
        acc_ref[slot] = blk
        pltpu.make_async_copy(
            acc_ref.at[slot],
            out_hbm.at[pl.ds(i * VBLK, VBLK), :],
            sems.at[slot],
        ).start()

    @pl.when(i == NVBLK - 1)
    def _():
        tail_ref[...] = blk[:TAIL]
        pltpu.make_async_copy(
            tail_ref,
            out_hbm.at[pl.ds((NVBLK - 1) * VBLK, TAIL), :],
            tail_sem,
        ).start()
        for k in range(NBUF - 1):
            step = NVBLK - NBUF + k
            pltpu.make_async_copy(
                acc_ref.at[step % NBUF],
                out_hbm.at[pl.ds(step * VBLK, VBLK), :],
                sems.at[step % NBUF],
            ).wait()
        pltpu.make_async_copy(
            tail_ref,
            out_hbm.at[pl.ds((NVBLK - 1) * VBLK, TAIL), :],
            tail_sem,
        ).wait()


@functools.cache
def _make_matmul():
    return pl.pallas_call(
        _mm_body,
        grid=(NVBLK,),
        in_specs=[
            pl.BlockSpec((B, D), lambda i: (0, 0)),
            pl.BlockSpec((D, VBLK), lambda i: (0, i)),
            pl.BlockSpec((1, 1, VBLK), lambda i: (i, 0, 0)),
        ],
        out_specs=pl.BlockSpec(memory_space=pltpu.HBM),
        out_shape=jax.ShapeDtypeStruct((VOCAB, B), jnp.float32),
        scratch_shapes=[
            pltpu.VMEM((NBUF, VBLK, B), jnp.float32),
            pltpu.VMEM((TAIL, B), jnp.float32),
            pltpu.SemaphoreType.DMA((NBUF,)),
            pltpu.SemaphoreType.DMA,
        ],
        compiler_params=pltpu.CompilerParams(
            dimension_semantics=("arbitrary",),
        ),
    )


def kernel(context_window, emb_table, W, b):
    idx = context_window.astype(jnp.int32).reshape(NW, BPW, CTX).transpose(0, 2, 1)
    mean_emb = _make_gather_mean()(idx, emb_table)
    b_pad = jnp.pad(b, (0, VPAD - VOCAB)).reshape(NVBLK, 1, VBLK)
    logits_t = _make_matmul()(mean_emb, W.T, b_pad)
    return logits_t.T
